# Initial kernel scaffold; baseline (speedup 1.0000x reference)
import jax
import jax.numpy as jnp
from jax.experimental import pallas as pl


def kernel(features, edge_index):
    raise NotImplementedError("write your pallas kernel here")


# speedup vs baseline: 7.2552x; 7.2552x over previous
import functools

import jax
import jax.numpy as jnp
from jax import lax
from jax.experimental import pallas as pl
from jax.experimental.pallas import tpu as pltpu
from jax.experimental.pallas import tpu_sc as plsc

N = 10000
D = 128
E = 320000
NLAYERS = 3

NC = 2
NS = 16
DH = D // NC

RPT = 632
N_PAD = NS * RPT

BLK = 128
EPT = -(-E // NS)
NBLK = -(-EPT // BLK)
E_PAD = NS * NBLK * BLK

SRC_PAD_ROW = N + 1
DST_PAD_ROW = N

_CHUNKS = []
_off = 0
while _off < RPT:
  _sz = min(BLK, RPT - _off)
  _CHUNKS.append((_off, _sz))
  _off += _sz

_mesh = plsc.VectorSubcoreMesh(core_axis_name="c", subcore_axis_name="s")


def _zero_rows(buf, nrows):
  zero = jnp.zeros((16,), jnp.float32)

  def zrow(i, carry):
    for j in range(DH // 16):
      buf[i, pl.ds(j * 16, 16)] = zero
    return carry

  lax.fori_loop(0, nrows, zrow, 0)


def _relu_rows(buf, nrows):
  zero = jnp.zeros((16,), jnp.float32)

  def rrow(i, carry):
    for j in range(DH // 16):
      buf[i, pl.ds(j * 16, 16)] = jnp.maximum(buf[i, pl.ds(j * 16, 16)], zero)
    return carry

  lax.fori_loop(0, nrows, rrow, 0)


@functools.partial(
    pl.kernel,
    out_type=jax.ShapeDtypeStruct((NC, N_PAD, DH), jnp.float32),
    mesh=_mesh,
    scratch_types=[
        pltpu.VMEM((NBLK, BLK), jnp.int32),
        pltpu.VMEM((NBLK, BLK), jnp.int32),
        pltpu.VMEM((BLK, DH), jnp.float32),
        pltpu.VMEM_SHARED((N_PAD, DH), jnp.float32),
        pltpu.VMEM_SHARED((N_PAD, DH), jnp.float32),
    ],
    compiler_params=pltpu.CompilerParams(use_tc_tiling_on_sc=False),
)
def _gcn_sc(f_hbm, src_hbm, dst_hbm, out_hbm, sidx, didx, gbuf, A, B):
  c = lax.axis_index("c")
  s = lax.axis_index("s")
  rbase = s * RPT

  pltpu.sync_copy(src_hbm.at[s], sidx)
  pltpu.sync_copy(dst_hbm.at[s], didx)
  pltpu.sync_copy(f_hbm.at[c, pl.ds(rbase, RPT)], A.at[pl.ds(rbase, RPT)])
  _zero_rows(gbuf, BLK)
  for off, sz in _CHUNKS:
    pltpu.sync_copy(gbuf.at[pl.ds(0, sz)], B.at[pl.ds(rbase + off, sz)])
  plsc.subcore_barrier()

  for layer in range(NLAYERS):
    def step(j, carry):
      pltpu.sync_copy(A.at[sidx.at[j]], gbuf)
      pltpu.sync_copy(gbuf, B.at[didx.at[j]], add=True)
      return carry

    lax.fori_loop(0, NBLK, step, 0)
    plsc.subcore_barrier()

    if layer < NLAYERS - 1:
      for off, sz in _CHUNKS:
        pltpu.sync_copy(B.at[pl.ds(rbase + off, sz)], gbuf.at[pl.ds(0, sz)])
        _relu_rows(gbuf, sz)
        pltpu.sync_copy(gbuf.at[pl.ds(0, sz)], A.at[pl.ds(rbase + off, sz)])
        _zero_rows(gbuf, sz)
        pltpu.sync_copy(gbuf.at[pl.ds(0, sz)], B.at[pl.ds(rbase + off, sz)])
      plsc.subcore_barrier()
    else:
      pltpu.sync_copy(B.at[pl.ds(rbase, RPT)],
                      out_hbm.at[c, pl.ds(rbase, RPT)])


def kernel(features, edge_index):
  src = edge_index[0].astype(jnp.int32)
  dst = edge_index[1].astype(jnp.int32)
  pad = E_PAD - E
  src = jnp.concatenate([src, jnp.full((pad,), SRC_PAD_ROW, jnp.int32)])
  dst = jnp.concatenate([dst, jnp.full((pad,), DST_PAD_ROW, jnp.int32)])
  src = src.reshape(NS, NBLK, BLK)
  dst = dst.reshape(NS, NBLK, BLK)

  f = jnp.concatenate(
      [features, jnp.zeros((N_PAD - N, D), jnp.float32)]
  ).reshape(N_PAD, NC, DH).transpose(1, 0, 2)

  out = _gcn_sc(f, src, dst)
  return out.transpose(1, 0, 2).reshape(N_PAD, D)[:N]


# speedup vs baseline: 10.2732x; 1.4160x over previous
import functools

import jax
import jax.numpy as jnp
from jax import lax
from jax.experimental import pallas as pl
from jax.experimental.pallas import tpu as pltpu
from jax.experimental.pallas import tpu_sc as plsc

N = 10000
D = 128
E = 320000
NLAYERS = 3

NC = 2
NS = 16
DH = D // NC

RPT = 632
N_PAD = NS * RPT

BLK = 64
EPT = -(-E // NS)
NBLK = -(-EPT // BLK)
E_PAD = NS * NBLK * BLK

SRC_PAD_ROW = N + 1
DST_PAD_ROW = N

_CHUNKS = []
_off = 0
while _off < RPT:
  _sz = min(BLK, RPT - _off)
  _CHUNKS.append((_off, _sz))
  _off += _sz

_mesh = plsc.VectorSubcoreMesh(core_axis_name="c", subcore_axis_name="s")


def _zero_rows(buf, nrows):
  zero = jnp.zeros((16,), jnp.float32)

  def zrow(i, carry):
    for j in range(DH // 16):
      buf[i, pl.ds(j * 16, 16)] = zero
    return carry

  lax.fori_loop(0, nrows, zrow, 0)


def _relu_rows(buf, nrows):
  zero = jnp.zeros((16,), jnp.float32)

  def rrow(i, carry):
    for j in range(DH // 16):
      buf[i, pl.ds(j * 16, 16)] = jnp.maximum(buf[i, pl.ds(j * 16, 16)], zero)
    return carry

  lax.fori_loop(0, nrows, rrow, 0)


@functools.partial(
    pl.kernel,
    out_type=jax.ShapeDtypeStruct((NC, N_PAD, DH), jnp.float32),
    mesh=_mesh,
    scratch_types=[
        pltpu.VMEM((NBLK, BLK), jnp.int32),
        pltpu.VMEM((NBLK, BLK), jnp.int32),
        pltpu.VMEM((2, BLK, DH), jnp.float32),
        pltpu.VMEM_SHARED((N_PAD, DH), jnp.float32),
        pltpu.VMEM_SHARED((N_PAD, DH), jnp.float32),
        pltpu.SemaphoreType.DMA,
        pltpu.SemaphoreType.DMA,
    ],
    compiler_params=pltpu.CompilerParams(use_tc_tiling_on_sc=False),
)
def _gcn_sc(f_hbm, src_hbm, dst_hbm, out_hbm, sidx, didx, gbuf, A, B,
            gsem, ssem):
  c = lax.axis_index("c")
  s = lax.axis_index("s")
  rbase = s * RPT

  pltpu.sync_copy(src_hbm.at[s], sidx)
  pltpu.sync_copy(dst_hbm.at[s], didx)
  pltpu.sync_copy(f_hbm.at[c, pl.ds(rbase, RPT)], A.at[pl.ds(rbase, RPT)])
  _zero_rows(gbuf.at[0], BLK)
  for off, sz in _CHUNKS:
    pltpu.sync_copy(gbuf.at[0, pl.ds(0, sz)], B.at[pl.ds(rbase + off, sz)])
  plsc.subcore_barrier()

  for layer in range(NLAYERS):
    pltpu.async_copy(A.at[sidx.at[0]], gbuf.at[0], gsem)

    def step(j, carry):
      b = lax.rem(j, 2)
      nb = lax.rem(j + 1, 2)

      @pl.when(j >= 1)
      def _():
        pltpu.make_async_copy(gbuf.at[nb], B.at[didx.at[j - 1]], ssem).wait()

      @pl.when(j + 1 < NBLK)
      def _():
        pltpu.async_copy(A.at[sidx.at[j + 1]], gbuf.at[nb], gsem)

      pltpu.make_async_copy(A.at[sidx.at[j]], gbuf.at[b], gsem).wait()
      pltpu.async_copy(gbuf.at[b], B.at[didx.at[j]], ssem, add=True)
      return carry

    lax.fori_loop(0, NBLK, step, 0)
    lastb = (NBLK - 1) % 2
    pltpu.make_async_copy(
        gbuf.at[lastb], B.at[didx.at[NBLK - 1]], ssem).wait()
    plsc.subcore_barrier()

    if layer < NLAYERS - 1:
      for off, sz in _CHUNKS:
        pltpu.sync_copy(B.at[pl.ds(rbase + off, sz)], gbuf.at[0, pl.ds(0, sz)])
        _relu_rows(gbuf.at[0], sz)
        pltpu.sync_copy(gbuf.at[0, pl.ds(0, sz)], A.at[pl.ds(rbase + off, sz)])
        _zero_rows(gbuf.at[0], sz)
        pltpu.sync_copy(gbuf.at[0, pl.ds(0, sz)], B.at[pl.ds(rbase + off, sz)])
      plsc.subcore_barrier()
    else:
      pltpu.sync_copy(B.at[pl.ds(rbase, RPT)],
                      out_hbm.at[c, pl.ds(rbase, RPT)])


def kernel(features, edge_index):
  src = edge_index[0].astype(jnp.int32)
  dst = edge_index[1].astype(jnp.int32)
  pad = E_PAD - E
  src = jnp.concatenate([src, jnp.full((pad,), SRC_PAD_ROW, jnp.int32)])
  dst = jnp.concatenate([dst, jnp.full((pad,), DST_PAD_ROW, jnp.int32)])
  src = src.reshape(NS, NBLK, BLK)
  dst = dst.reshape(NS, NBLK, BLK)

  f = jnp.concatenate(
      [features, jnp.zeros((N_PAD - N, D), jnp.float32)]
  ).reshape(N_PAD, NC, DH).transpose(1, 0, 2)

  out = _gcn_sc(f, src, dst)
  return out.transpose(1, 0, 2).reshape(N_PAD, D)[:N]


# speedup vs baseline: 11.5218x; 1.1215x over previous
import functools

import jax
import jax.numpy as jnp
from jax import lax
from jax.experimental import pallas as pl
from jax.experimental.pallas import tpu as pltpu
from jax.experimental.pallas import tpu_sc as plsc

N = 10000
D = 128
E = 320000
NLAYERS = 3

NC = 2
NS = 16
DH = D // NC

RPT = 632
N_PAD = NS * RPT

BLK = 64
EPT = -(-E // NS)
NBLK = -(-EPT // BLK)
E_PAD = NS * NBLK * BLK

SRC_PAD_ROW = N + 1
DST_PAD_ROW = N

_CHUNKS = []
_off = 0
while _off < RPT:
  _sz = min(BLK, RPT - _off)
  _CHUNKS.append((_off, _sz))
  _off += _sz

_mesh = plsc.VectorSubcoreMesh(core_axis_name="c", subcore_axis_name="s")


def _zero_rows(buf, nrows):
  zero = jnp.zeros((16,), jnp.float32)

  def zrow(i, carry):
    for j in range(DH // 16):
      buf[i, pl.ds(j * 16, 16)] = zero
    return carry

  lax.fori_loop(0, nrows, zrow, 0)


def _relu_rows(buf, nrows):
  zero = jnp.zeros((16,), jnp.float32)

  def rrow(i, carry):
    for j in range(DH // 16):
      buf[i, pl.ds(j * 16, 16)] = jnp.maximum(buf[i, pl.ds(j * 16, 16)], zero)
    return carry

  lax.fori_loop(0, nrows, rrow, 0)


FL = N // NS


@functools.partial(
    pl.kernel,
    out_type=jax.ShapeDtypeStruct((N, D), jnp.float32),
    mesh=_mesh,
    scratch_types=[
        pltpu.VMEM((NBLK, BLK), jnp.int32),
        pltpu.VMEM((NBLK, BLK), jnp.int32),
        pltpu.VMEM((2, BLK, DH), jnp.float32),
        pltpu.VMEM_SHARED((N_PAD, DH), jnp.float32),
        pltpu.VMEM_SHARED((N_PAD, DH), jnp.float32),
        pltpu.SemaphoreType.DMA,
        pltpu.SemaphoreType.DMA,
    ],
    compiler_params=pltpu.CompilerParams(use_tc_tiling_on_sc=False),
)
def _gcn_sc(f_hbm, src_hbm, dst_hbm, out_hbm, sidx, didx, gbuf, A, B,
            gsem, ssem):
  c = lax.axis_index("c")
  s = lax.axis_index("s")
  rbase = s * RPT

  pltpu.sync_copy(src_hbm.at[s], sidx)
  pltpu.sync_copy(dst_hbm.at[s], didx)
  pltpu.sync_copy(f_hbm.at[pl.ds(s * FL, FL), pl.ds(c * DH, DH)],
                  A.at[pl.ds(s * FL, FL)])
  _zero_rows(gbuf.at[0], BLK)
  for off, sz in _CHUNKS:
    pltpu.sync_copy(gbuf.at[0, pl.ds(0, sz)], B.at[pl.ds(rbase + off, sz)])

  @pl.when(s == 0)
  def _():
    done = 0
    while N + done < N_PAD:
      sz = min(BLK, N_PAD - N - done)
      pltpu.sync_copy(gbuf.at[0, pl.ds(0, sz)], A.at[pl.ds(N + done, sz)])
      done += sz

  plsc.subcore_barrier()

  for layer in range(NLAYERS):
    pltpu.async_copy(A.at[sidx.at[0]], gbuf.at[0], gsem)

    def step(j, carry):
      b = lax.rem(j, 2)
      nb = lax.rem(j + 1, 2)

      @pl.when(j >= 1)
      def _():
        pltpu.make_async_copy(gbuf.at[nb], B.at[didx.at[j - 1]], ssem).wait()

      @pl.when(j + 1 < NBLK)
      def _():
        pltpu.async_copy(A.at[sidx.at[j + 1]], gbuf.at[nb], gsem)

      pltpu.make_async_copy(A.at[sidx.at[j]], gbuf.at[b], gsem).wait()
      pltpu.async_copy(gbuf.at[b], B.at[didx.at[j]], ssem, add=True)
      return carry

    lax.fori_loop(0, NBLK, step, 0)
    lastb = (NBLK - 1) % 2
    pltpu.make_async_copy(
        gbuf.at[lastb], B.at[didx.at[NBLK - 1]], ssem).wait()
    plsc.subcore_barrier()

    if layer < NLAYERS - 1:
      for off, sz in _CHUNKS:
        pltpu.sync_copy(B.at[pl.ds(rbase + off, sz)], gbuf.at[0, pl.ds(0, sz)])
        _relu_rows(gbuf.at[0], sz)
        pltpu.sync_copy(gbuf.at[0, pl.ds(0, sz)], A.at[pl.ds(rbase + off, sz)])
        _zero_rows(gbuf.at[0], sz)
        pltpu.sync_copy(gbuf.at[0, pl.ds(0, sz)], B.at[pl.ds(rbase + off, sz)])
      plsc.subcore_barrier()
    else:
      pltpu.sync_copy(B.at[pl.ds(s * FL, FL)],
                      out_hbm.at[pl.ds(s * FL, FL), pl.ds(c * DH, DH)])


def kernel(features, edge_index):
  src = edge_index[0].astype(jnp.int32)
  dst = edge_index[1].astype(jnp.int32)
  pad = E_PAD - E
  src = jnp.concatenate([src, jnp.full((pad,), SRC_PAD_ROW, jnp.int32)])
  dst = jnp.concatenate([dst, jnp.full((pad,), DST_PAD_ROW, jnp.int32)])
  src = src.reshape(NS, NBLK, BLK)
  dst = dst.reshape(NS, NBLK, BLK)

  return _gcn_sc(features, src, dst)
